# baseline (speedup 1.0000x reference)
import jax
import jax.numpy as jnp
from jax import lax
from jax.experimental import pallas as pl
from jax.experimental.pallas import tpu as pltpu
from jax.experimental.pallas import tpu_sc as plsc

N = 10000
E = 320000
NC = 2
NS = 16
NW = NC * NS
EPW = E // NW
NP = 10240
RPT = NP // NS

_mesh = plsc.VectorSubcoreMesh(core_axis_name="c", subcore_axis_name="s")


def _make_agg(d, with_count, C):
    T = EPW // C
    out_type = [jax.ShapeDtypeStruct((NC, NP, d), jnp.float32)]
    scratch = [
        pltpu.VMEM((EPW,), jnp.int32),
        pltpu.VMEM((EPW,), jnp.int32),
        pltpu.VMEM((C, d), jnp.float32),
        pltpu.VMEM((C, d), jnp.float32),
        pltpu.VMEM_SHARED((NP, d), jnp.float32),
        pltpu.SemaphoreType.DMA,
        pltpu.SemaphoreType.DMA,
        pltpu.SemaphoreType.DMA,
        pltpu.SemaphoreType.DMA,
    ]
    if with_count:
        out_type.append(jax.ShapeDtypeStruct((NW, NP), jnp.float32))
        scratch += [
            pltpu.VMEM((NP,), jnp.float32),
        ]

    def body(*refs):
        if with_count:
            (eidx_hbm, y_hbm, z_hbm, p_hbm, pc_hbm,
             srcv, dstv, bufa, bufb, acc, sema, semb,
             ssema, ssemb, hist) = refs
        else:
            (eidx_hbm, y_hbm, z_hbm, p_hbm,
             srcv, dstv, bufa, bufb, acc, sema, semb,
             ssema, ssemb) = refs
        cid = lax.axis_index("c")
        sid = lax.axis_index("s")
        wid = sid * NC + cid
        e0 = wid * EPW
        pltpu.sync_copy(eidx_hbm.at[pl.ds(e0, EPW)], srcv)
        pltpu.sync_copy(eidx_hbm.at[pl.ds(E + e0, EPW)], dstv)
        r0 = sid * RPT
        pltpu.sync_copy(z_hbm.at[pl.ds(r0, RPT)], acc.at[pl.ds(r0, RPT)])
        if with_count:
            def zero_hist(i, carry):
                hist[pl.ds(i * 16, 16)] = jnp.zeros((16,), jnp.float32)
                return carry
            lax.fori_loop(0, NP // 16, zero_hist, 0)
        plsc.subcore_barrier()

        def chunk(t, buf, sem, ssem, nbuf, nsem, nssem):
            @pl.when(t >= 1)
            def _():
                pltpu.make_async_copy(
                    nbuf, acc.at[dstv.at[pl.ds((t - 1) * C, C)]],
                    nssem).wait()

            @pl.when(t + 1 < T)
            def _():
                pltpu.async_copy(
                    y_hbm.at[srcv.at[pl.ds((t + 1) * C, C)]], nbuf, nsem)
            if with_count:
                ones16 = jnp.ones((16,), jnp.float32)

                def hstep(i, carry):
                    idx = dstv[pl.ds(t * C + i * 16, 16)]
                    plsc.addupdate_scatter(hist, [idx], ones16)
                    return carry
                lax.fori_loop(0, C // 16, hstep, 0)
            pltpu.make_async_copy(
                y_hbm.at[srcv.at[pl.ds(t * C, C)]], buf, sem).wait()
            pltpu.async_copy(
                buf, acc.at[dstv.at[pl.ds(t * C, C)]], ssem, add=True)

        pltpu.async_copy(y_hbm.at[srcv.at[pl.ds(0, C)]], bufa, sema)

        def step(t, carry):
            @pl.when(t % 2 == 0)
            def _():
                chunk(t, bufa, sema, ssema, bufb, semb, ssemb)

            @pl.when(t % 2 == 1)
            def _():
                chunk(t, bufb, semb, ssemb, bufa, sema, ssema)
            return carry

        lax.fori_loop(0, T, step, 0)
        bufs = (bufa, bufb)
        ssems = (ssema, ssemb)
        ia = (T - 1) % 2
        pltpu.make_async_copy(
            bufs[ia], acc.at[dstv.at[pl.ds((T - 1) * C, C)]],
            ssems[ia]).wait()
        if with_count:
            pltpu.sync_copy(hist, pc_hbm.at[wid])
        plsc.subcore_barrier()
        pltpu.sync_copy(acc.at[pl.ds(r0, RPT)], p_hbm.at[cid, pl.ds(r0, RPT)])

    return pl.kernel(body, out_type=out_type if with_count else out_type[0],
                     mesh=_mesh, scratch_types=scratch,
                     compiler_params=pltpu.CompilerParams(
                         use_tc_tiling_on_sc=False,
                         needs_layout_passes=False))


_agg64c = _make_agg(64, True, 400)
_agg32 = _make_agg(32, False, 1000)
_agg16 = _make_agg(16, False, 2000)

R = 2048


def _dense_one(x, w, b):
    d_in, d_out = w.shape

    def body(x_ref, w_ref, b_ref, y_ref):
        y_ref[...] = jnp.dot(x_ref[...], w_ref[...],
                             preferred_element_type=jnp.float32) + b_ref[...]

    return pl.pallas_call(
        body,
        grid=(pl.cdiv(N, R),),
        in_specs=[
            pl.BlockSpec((R, d_in), lambda i: (i, 0)),
            pl.BlockSpec((d_in, d_out), lambda i: (0, 0)),
            pl.BlockSpec((1, d_out), lambda i: (0, 0)),
        ],
        out_specs=pl.BlockSpec((R, d_out), lambda i: (i, 0)),
        out_shape=jax.ShapeDtypeStruct((N, d_out), jnp.float32),
    )(x, w, b.reshape(1, -1))


def _comb1_yl(p, pc, yr, wl):
    d = yr.shape[1]
    d_out = wl.shape[1]

    def body(p_ref, pc_ref, yr_ref, wl_ref, yl_o, inv_o):
        s = p_ref[0] + p_ref[1]
        cnt = jnp.sum(pc_ref[...], axis=0)
        inv1 = 1.0 / jnp.maximum(cnt, 1.0)
        inv = jnp.reshape(inv1, (R, 1))
        h = jnp.maximum(s * inv + yr_ref[...], 0.0)
        yl_o[...] = jnp.dot(h, wl_ref[...],
                            preferred_element_type=jnp.float32)
        inv_o[...] = inv1

    return pl.pallas_call(
        body,
        grid=(pl.cdiv(N, R),),
        in_specs=[
            pl.BlockSpec((NC, R, d), lambda i: (0, i, 0)),
            pl.BlockSpec((NW, R), lambda i: (0, i)),
            pl.BlockSpec((R, d), lambda i: (i, 0)),
            pl.BlockSpec((d, d_out), lambda i: (0, 0)),
        ],
        out_specs=[pl.BlockSpec((R, d_out), lambda i: (i, 0)),
                   pl.BlockSpec((R,), lambda i: (i,))],
        out_shape=[jax.ShapeDtypeStruct((N, d_out), jnp.float32),
                   jax.ShapeDtypeStruct((N,), jnp.float32)],
    )(p, pc, yr, wl)


def _comb_yl(p, inv, yr, wl):
    d = yr.shape[1]
    d_out = wl.shape[1]

    def body(p_ref, inv_ref, yr_ref, wl_ref, yl_o):
        s = p_ref[0] + p_ref[1]
        inv = jnp.reshape(inv_ref[...], (R, 1))
        h = jnp.maximum(s * inv + yr_ref[...], 0.0)
        yl_o[...] = jnp.dot(h, wl_ref[...],
                            preferred_element_type=jnp.float32)

    return pl.pallas_call(
        body,
        grid=(pl.cdiv(N, R),),
        in_specs=[
            pl.BlockSpec((NC, R, d), lambda i: (0, i, 0)),
            pl.BlockSpec((R,), lambda i: (i,)),
            pl.BlockSpec((R, d), lambda i: (i, 0)),
            pl.BlockSpec((d, d_out), lambda i: (0, 0)),
        ],
        out_specs=pl.BlockSpec((R, d_out), lambda i: (i, 0)),
        out_shape=jax.ShapeDtypeStruct((N, d_out), jnp.float32),
    )(p, inv, yr, wl)


def _comb_yr(p, inv, yr, wr, b):
    d = yr.shape[1]
    d_out = wr.shape[1]

    def body(p_ref, inv_ref, yr_ref, wr_ref, b_ref, yr_o):
        s = p_ref[0] + p_ref[1]
        inv = jnp.reshape(inv_ref[...], (R, 1))
        h = jnp.maximum(s * inv + yr_ref[...], 0.0)
        yr_o[...] = jnp.dot(h, wr_ref[...],
                            preferred_element_type=jnp.float32) + b_ref[...]

    return pl.pallas_call(
        body,
        grid=(pl.cdiv(N, R),),
        in_specs=[
            pl.BlockSpec((NC, R, d), lambda i: (0, i, 0)),
            pl.BlockSpec((R,), lambda i: (i,)),
            pl.BlockSpec((R, d), lambda i: (i, 0)),
            pl.BlockSpec((d, d_out), lambda i: (0, 0)),
            pl.BlockSpec((1, d_out), lambda i: (0, 0)),
        ],
        out_specs=pl.BlockSpec((R, d_out), lambda i: (i, 0)),
        out_shape=jax.ShapeDtypeStruct((N, d_out), jnp.float32),
    )(p, inv, yr, wr, b.reshape(1, -1))


def _combine_last(p, inv, yr):
    d = yr.shape[1]

    def body(p_ref, inv_ref, yr_ref, out_o):
        inv = jnp.reshape(inv_ref[...], (R, 1))
        full = (p_ref[0] + p_ref[1]) * inv + yr_ref[...]
        out_o[...] = full[:, :10]

    return pl.pallas_call(
        body,
        grid=(pl.cdiv(N, R),),
        in_specs=[
            pl.BlockSpec((NC, R, d), lambda i: (0, i, 0)),
            pl.BlockSpec((R,), lambda i: (i,)),
            pl.BlockSpec((R, d), lambda i: (i, 0)),
        ],
        out_specs=pl.BlockSpec((R, 10), lambda i: (i, 0)),
        out_shape=jax.ShapeDtypeStruct((N, 10), jnp.float32),
    )(p, inv, yr)


def kernel(x, edge_index, edge_weight,
           W1l, W1r, b1, W2l, W2r, b2, W3l, W3r, b3, W4l, W4r, b4):
    del edge_weight
    eidx = edge_index.astype(jnp.int32).reshape(2 * E)
    z64 = jnp.zeros((NP, 64), jnp.float32)
    z32 = jnp.zeros((NP, 32), jnp.float32)
    z16 = jnp.zeros((NP, 16), jnp.float32)
    zb = jnp.zeros((64,), jnp.float32)
    W4l_p = jnp.pad(W4l, ((0, 0), (0, 6)))
    W4r_p = jnp.pad(W4r, ((0, 0), (0, 6)))
    b4_p = jnp.pad(b4, (0, 6))

    yl1 = _dense_one(x, W1l, zb)
    p1, pc = _agg64c(eidx, yl1, z64)
    yr1 = _dense_one(x, W1r, b1)
    yl2, inv = _comb1_yl(p1, pc, yr1, W2l)
    p2 = _agg32(eidx, yl2, z32)
    yr2 = _comb_yr(p1, inv, yr1, W2r, b2)
    yl3 = _comb_yl(p2, inv, yr2, W3l)
    p3 = _agg16(eidx, yl3, z16)
    yr3 = _comb_yr(p2, inv, yr2, W3r, b3)
    yl4 = _comb_yl(p3, inv, yr3, W4l_p)
    p4 = _agg16(eidx, yl4, z16)
    yr4 = _comb_yr(p3, inv, yr3, W4r_p, b4_p)
    return _combine_last(p4, inv, yr4)


# speedup vs baseline: 1.0262x; 1.0262x over previous
import jax
import jax.numpy as jnp
from jax import lax
from jax.experimental import pallas as pl
from jax.experimental.pallas import tpu as pltpu
from jax.experimental.pallas import tpu_sc as plsc

N = 10000
E = 320000
NC = 2
NS = 16
NW = NC * NS
EPW = E // NW
NP = 10240
RPT = NP // NS

_mesh = plsc.VectorSubcoreMesh(core_axis_name="c", subcore_axis_name="s")


def _make_agg(d, with_count, C):
    T = EPW // C
    out_type = [jax.ShapeDtypeStruct((NC, NP, d), jnp.float32)]
    scratch = [
        pltpu.VMEM((EPW,), jnp.int32),
        pltpu.VMEM((EPW,), jnp.int32),
        pltpu.VMEM((C, d), jnp.float32),
        pltpu.VMEM((C, d), jnp.float32),
        pltpu.VMEM_SHARED((NP, d), jnp.float32),
        pltpu.SemaphoreType.DMA,
        pltpu.SemaphoreType.DMA,
        pltpu.SemaphoreType.DMA,
        pltpu.SemaphoreType.DMA,
    ]
    if with_count:
        out_type.append(jax.ShapeDtypeStruct((NW, NP), jnp.float32))
        scratch += [
            pltpu.VMEM((NP,), jnp.float32),
        ]

    def body(*refs):
        if with_count:
            (eidx_hbm, y_hbm, z_hbm, p_hbm, pc_hbm,
             srcv, dstv, bufa, bufb, acc, sema, semb,
             ssema, ssemb, hist) = refs
        else:
            (eidx_hbm, y_hbm, z_hbm, p_hbm,
             srcv, dstv, bufa, bufb, acc, sema, semb,
             ssema, ssemb) = refs
        cid = lax.axis_index("c")
        sid = lax.axis_index("s")
        wid = sid * NC + cid
        e0 = wid * EPW
        pltpu.sync_copy(eidx_hbm.at[pl.ds(e0, EPW)], srcv)
        pltpu.sync_copy(eidx_hbm.at[pl.ds(E + e0, EPW)], dstv)
        r0 = sid * RPT
        pltpu.sync_copy(z_hbm.at[pl.ds(r0, RPT)], acc.at[pl.ds(r0, RPT)])
        if with_count:
            def zero_hist(i, carry):
                hist[pl.ds(i * 16, 16)] = jnp.zeros((16,), jnp.float32)
                return carry
            lax.fori_loop(0, NP // 16, zero_hist, 0)
        plsc.subcore_barrier()

        def chunk(t, buf, sem, ssem, nbuf, nsem, nssem):
            @pl.when(t >= 1)
            def _():
                pltpu.make_async_copy(
                    nbuf, acc.at[dstv.at[pl.ds((t - 1) * C, C)]],
                    nssem).wait()

            @pl.when(t + 1 < T)
            def _():
                pltpu.async_copy(
                    y_hbm.at[srcv.at[pl.ds((t + 1) * C, C)]], nbuf, nsem)
            if with_count:
                ones16 = jnp.ones((16,), jnp.float32)

                def hstep(i, carry):
                    idx = dstv[pl.ds(t * C + i * 16, 16)]
                    plsc.addupdate_scatter(hist, [idx], ones16)
                    return carry
                lax.fori_loop(0, C // 16, hstep, 0)
            pltpu.make_async_copy(
                y_hbm.at[srcv.at[pl.ds(t * C, C)]], buf, sem).wait()
            pltpu.async_copy(
                buf, acc.at[dstv.at[pl.ds(t * C, C)]], ssem, add=True)

        pltpu.async_copy(y_hbm.at[srcv.at[pl.ds(0, C)]], bufa, sema)

        def step(t, carry):
            @pl.when(t % 2 == 0)
            def _():
                chunk(t, bufa, sema, ssema, bufb, semb, ssemb)

            @pl.when(t % 2 == 1)
            def _():
                chunk(t, bufb, semb, ssemb, bufa, sema, ssema)
            return carry

        lax.fori_loop(0, T, step, 0)
        bufs = (bufa, bufb)
        ssems = (ssema, ssemb)
        ia = (T - 1) % 2
        pltpu.make_async_copy(
            bufs[ia], acc.at[dstv.at[pl.ds((T - 1) * C, C)]],
            ssems[ia]).wait()
        if with_count:
            pltpu.sync_copy(hist, pc_hbm.at[wid])
        plsc.subcore_barrier()
        pltpu.sync_copy(acc.at[pl.ds(r0, RPT)], p_hbm.at[cid, pl.ds(r0, RPT)])

    return pl.kernel(body, out_type=out_type if with_count else out_type[0],
                     mesh=_mesh, scratch_types=scratch,
                     compiler_params=pltpu.CompilerParams(
                         use_tc_tiling_on_sc=False,
                         needs_layout_passes=False))


_agg64c = _make_agg(64, True, 400)
_agg32 = _make_agg(32, False, 1000)
_agg16 = _make_agg(16, False, 2000)

R = 2048


def _dense0(x, wl, wr, b):
    d_in, d_out = wl.shape

    def body(x_ref, wl_ref, wr_ref, b_ref, yl_ref, yr_ref):
        xb = x_ref[...]
        yl_ref[...] = jnp.dot(xb, wl_ref[...],
                              preferred_element_type=jnp.float32)
        yr_ref[...] = jnp.dot(xb, wr_ref[...],
                              preferred_element_type=jnp.float32) + b_ref[...]

    return pl.pallas_call(
        body,
        grid=(pl.cdiv(N, R),),
        in_specs=[
            pl.BlockSpec((R, d_in), lambda i: (i, 0)),
            pl.BlockSpec((d_in, d_out), lambda i: (0, 0)),
            pl.BlockSpec((d_in, d_out), lambda i: (0, 0)),
            pl.BlockSpec((1, d_out), lambda i: (0, 0)),
        ],
        out_specs=[pl.BlockSpec((R, d_out), lambda i: (i, 0)),
                   pl.BlockSpec((R, d_out), lambda i: (i, 0))],
        out_shape=[jax.ShapeDtypeStruct((N, d_out), jnp.float32)] * 2,
    )(x, wl, wr, b.reshape(1, -1))


def _combine1(p, pc, yr, wl, wr, b):
    d = yr.shape[1]
    d_out = wl.shape[1]

    def body(p_ref, pc_ref, yr_ref, wl_ref, wr_ref, b_ref,
             yl_o, yr_o, inv_o):
        s = p_ref[0] + p_ref[1]
        cnt = jnp.sum(pc_ref[...], axis=0)
        inv1 = 1.0 / jnp.maximum(cnt, 1.0)
        inv = jnp.reshape(inv1, (R, 1))
        h = jnp.maximum(s * inv + yr_ref[...], 0.0)
        yl_o[...] = jnp.dot(h, wl_ref[...],
                            preferred_element_type=jnp.float32)
        yr_o[...] = jnp.dot(h, wr_ref[...],
                            preferred_element_type=jnp.float32) + b_ref[...]
        inv_o[...] = inv1

    return pl.pallas_call(
        body,
        grid=(pl.cdiv(N, R),),
        in_specs=[
            pl.BlockSpec((NC, R, d), lambda i: (0, i, 0)),
            pl.BlockSpec((NW, R), lambda i: (0, i)),
            pl.BlockSpec((R, d), lambda i: (i, 0)),
            pl.BlockSpec((d, d_out), lambda i: (0, 0)),
            pl.BlockSpec((d, d_out), lambda i: (0, 0)),
            pl.BlockSpec((1, d_out), lambda i: (0, 0)),
        ],
        out_specs=[pl.BlockSpec((R, d_out), lambda i: (i, 0)),
                   pl.BlockSpec((R, d_out), lambda i: (i, 0)),
                   pl.BlockSpec((R,), lambda i: (i,))],
        out_shape=[jax.ShapeDtypeStruct((N, d_out), jnp.float32),
                   jax.ShapeDtypeStruct((N, d_out), jnp.float32),
                   jax.ShapeDtypeStruct((N,), jnp.float32)],
    )(p, pc, yr, wl, wr, b.reshape(1, -1))


def _combine_mid(p, inv, yr, wl, wr, b):
    d = yr.shape[1]
    d_out = wl.shape[1]

    def body(p_ref, inv_ref, yr_ref, wl_ref, wr_ref, b_ref, yl_o, yr_o):
        s = p_ref[0] + p_ref[1]
        inv = jnp.reshape(inv_ref[...], (R, 1))
        h = jnp.maximum(s * inv + yr_ref[...], 0.0)
        yl_o[...] = jnp.dot(h, wl_ref[...],
                            preferred_element_type=jnp.float32)
        yr_o[...] = jnp.dot(h, wr_ref[...],
                            preferred_element_type=jnp.float32) + b_ref[...]

    return pl.pallas_call(
        body,
        grid=(pl.cdiv(N, R),),
        in_specs=[
            pl.BlockSpec((NC, R, d), lambda i: (0, i, 0)),
            pl.BlockSpec((R,), lambda i: (i,)),
            pl.BlockSpec((R, d), lambda i: (i, 0)),
            pl.BlockSpec((d, d_out), lambda i: (0, 0)),
            pl.BlockSpec((d, d_out), lambda i: (0, 0)),
            pl.BlockSpec((1, d_out), lambda i: (0, 0)),
        ],
        out_specs=[pl.BlockSpec((R, d_out), lambda i: (i, 0)),
                   pl.BlockSpec((R, d_out), lambda i: (i, 0))],
        out_shape=[jax.ShapeDtypeStruct((N, d_out), jnp.float32),
                   jax.ShapeDtypeStruct((N, d_out), jnp.float32)],
    )(p, inv, yr, wl, wr, b.reshape(1, -1))


def _combine_last(p, inv, yr):
    d = yr.shape[1]

    def body(p_ref, inv_ref, yr_ref, out_o):
        inv = jnp.reshape(inv_ref[...], (R, 1))
        full = (p_ref[0] + p_ref[1]) * inv + yr_ref[...]
        out_o[...] = full[:, :10]

    return pl.pallas_call(
        body,
        grid=(pl.cdiv(N, R),),
        in_specs=[
            pl.BlockSpec((NC, R, d), lambda i: (0, i, 0)),
            pl.BlockSpec((R,), lambda i: (i,)),
            pl.BlockSpec((R, d), lambda i: (i, 0)),
        ],
        out_specs=pl.BlockSpec((R, 10), lambda i: (i, 0)),
        out_shape=jax.ShapeDtypeStruct((N, 10), jnp.float32),
    )(p, inv, yr)


def kernel(x, edge_index, edge_weight,
           W1l, W1r, b1, W2l, W2r, b2, W3l, W3r, b3, W4l, W4r, b4):
    del edge_weight
    eidx = edge_index.astype(jnp.int32).reshape(2 * E)
    z64 = jnp.zeros((NP, 64), jnp.float32)
    z32 = jnp.zeros((NP, 32), jnp.float32)
    z16 = jnp.zeros((NP, 16), jnp.float32)
    W4l_p = jnp.pad(W4l, ((0, 0), (0, 6)))
    W4r_p = jnp.pad(W4r, ((0, 0), (0, 6)))
    b4_p = jnp.pad(b4, (0, 6))

    yl1, yr1 = _dense0(x, W1l, W1r, b1)
    p1, pc = _agg64c(eidx, yl1, z64)
    yl2, yr2, inv = _combine1(p1, pc, yr1, W2l, W2r, b2)
    p2 = _agg32(eidx, yl2, z32)
    yl3, yr3 = _combine_mid(p2, inv, yr2, W3l, W3r, b3)
    p3 = _agg16(eidx, yl3, z16)
    yl4, yr4 = _combine_mid(p3, inv, yr3, W4l_p, W4r_p, b4_p)
    p4 = _agg16(eidx, yl4, z16)
    return _combine_last(p4, inv, yr4)
